# Initial kernel scaffold; baseline (speedup 1.0000x reference)
import jax
import jax.numpy as jnp
from jax.experimental import pallas as pl


def kernel(surf_x, graph_x, bip_edge_weight, W_surf_in, W_graph_in, W_graph_blocks, W_graph2surf, W_surf2graph, W_out, graph_edge_index, bip_edge_index):
    raise NotImplementedError("write your pallas kernel here")


# speedup vs baseline: 3.1257x; 3.1257x over previous
import functools

import jax
import jax.numpy as jnp
from jax import lax
from jax.experimental import pallas as pl
from jax.experimental.pallas import tpu as pltpu
from jax.experimental.pallas import tpu_sc as plsc

NG = 10000
NS = 20000
EG = 320000
EB = 640000
N_BLOCK = 4
CH = 64
K = 128
NSUB = 16
NCORE = 2
RB = 1000


def _pad_edges(src, dst, w, n_src, n_dst):
  e = src.shape[0]
  epad = ((e + NSUB * K - 1) // (NSUB * K)) * (NSUB * K)
  pad = epad - e
  ar = jnp.arange(pad, dtype=jnp.int32)
  src_p = jnp.concatenate([src.astype(jnp.int32), ar % n_src])
  dst_p = jnp.concatenate([dst.astype(jnp.int32), ar % n_dst])
  w_p = jnp.concatenate([w, jnp.zeros((pad,), jnp.float32)])
  return src_p, dst_p, w_p, epad


@functools.cache
def _make_agg(n_src, n_dst, epad, clip_lo):
  e_per = epad // NSUB
  n_chunks = e_per // K
  per_sub = (n_dst // NSUB) & ~7
  rem = n_dst - NSUB * per_sub
  chunks = []
  off = 0
  while off < per_sub:
    chunks.append((off, min(K, per_sub - off)))
    off += min(K, per_sub - off)
  mesh = plsc.VectorSubcoreMesh(core_axis_name="c", subcore_axis_name="s")

  def body(x_hbm, src_hbm, dst_hbm, w_hbm, zk_hbm, z1_hbm, out_hbm,
           src_v, dst_v, w_v, den_v, rows_v, io_v, acc, acc1, sem):
    cid = lax.axis_index("c")
    sid = lax.axis_index("s")
    row0 = pl.multiple_of(sid * per_sub, 8)

    pltpu.sync_copy(zk_hbm, io_v)
    pltpu.sync_copy(z1_hbm, den_v)
    for off, sz in chunks:
      pltpu.sync_copy(io_v.at[pl.ds(0, sz)], acc.at[pl.ds(row0 + off, sz)])
      pltpu.sync_copy(den_v.at[pl.ds(0, sz)], acc1.at[pl.ds(row0 + off, sz)])
    if rem:
      @pl.when(sid == 0)
      def _():
        base = NSUB * per_sub
        pltpu.sync_copy(io_v.at[pl.ds(0, rem)], acc.at[pl.ds(base, rem)])
        pltpu.sync_copy(den_v.at[pl.ds(0, rem)], acc1.at[pl.ds(base, rem)])
    plsc.subcore_barrier()

    coff = cid * n_src

    def chunk(k, carry):
      base = sid * e_per + k * K
      pltpu.sync_copy(src_hbm.at[pl.ds(base, K)], src_v)
      pltpu.sync_copy(dst_hbm.at[pl.ds(base, K)], dst_v)
      pltpu.sync_copy(w_hbm.at[pl.ds(base, K)], w_v)
      for i in range(K // 16):
        sl = pl.ds(i * 16, 16)
        src_v[sl] = src_v[sl] + coff
      pltpu.async_copy(x_hbm.at[src_v], rows_v, sem).wait()

      def scale_row(r, c2):
        wb = plsc.load_gather(w_v, [jnp.full((16,), r, jnp.int32)])
        for c in range(CH // 16):
          sl = pl.ds(c * 16, 16)
          rows_v[r, sl] = rows_v[r, sl] * wb
        return c2

      lax.fori_loop(0, K, scale_row, 0)
      pltpu.sync_copy(rows_v, acc.at[dst_v], add=True)
      pltpu.sync_copy(w_v, acc1.at[dst_v], add=True)
      return carry

    lax.fori_loop(0, n_chunks, chunk, 0)
    plsc.subcore_barrier()

    def emit_readout(r0, sz):
      pltpu.sync_copy(acc.at[pl.ds(r0, sz)], io_v.at[pl.ds(0, sz)])
      pltpu.sync_copy(acc1.at[pl.ds(r0, sz)], den_v.at[pl.ds(0, sz)])

      def div_row(r, c2):
        db = plsc.load_gather(den_v, [jnp.full((16,), r, jnp.int32)])
        inv = 1.0 / jnp.maximum(db, clip_lo)
        for c in range(CH // 16):
          sl = pl.ds(c * 16, 16)
          io_v[r, sl] = io_v[r, sl] * inv
        return c2

      lax.fori_loop(0, sz, div_row, 0)
      pltpu.sync_copy(io_v.at[pl.ds(0, sz)],
                      out_hbm.at[pl.ds(pl.multiple_of(cid * n_dst + r0, 8), sz)])

    for off, sz in chunks:
      emit_readout(row0 + off, sz)
    if rem:
      @pl.when(sid == 0)
      def _():
        emit_readout(NSUB * per_sub, rem)

  return pl.kernel(
      body,
      out_type=jax.ShapeDtypeStruct((NCORE * n_dst, CH), jnp.float32),
      mesh=mesh,
      compiler_params=pltpu.CompilerParams(
          needs_layout_passes=False, use_tc_tiling_on_sc=False),
      scratch_types=[
          pltpu.VMEM((K,), jnp.int32),
          pltpu.VMEM((K,), jnp.int32),
          pltpu.VMEM((K,), jnp.float32),
          pltpu.VMEM((K,), jnp.float32),
          pltpu.VMEM((K, CH), jnp.float32),
          pltpu.VMEM((K, CH), jnp.float32),
          pltpu.VMEM_SHARED((n_dst, CH), jnp.float32),
          pltpu.VMEM_SHARED((n_dst,), jnp.float32),
          pltpu.SemaphoreType.DMA,
      ])


def _split_w(w):
  kdim = w.shape[0]
  return jnp.swapaxes(w.reshape(kdim, NCORE, CH), 0, 1).reshape(NCORE * kdim, CH)


def _tc_embed(x, ws, n, kdim):
  nb = n // RB

  def body(x_ref, w_ref, o_ref):
    o_ref[...] = lax.dot(x_ref[...], w_ref[...])

  return pl.pallas_call(
      body,
      grid=(NCORE, nb),
      in_specs=[
          pl.BlockSpec((RB, kdim), lambda c, b: (b, 0)),
          pl.BlockSpec((kdim, CH), lambda c, b: (c, 0)),
      ],
      out_specs=pl.BlockSpec((RB, CH), lambda c, b: (c * nb + b, 0)),
      out_shape=jax.ShapeDtypeStruct((NCORE * n, CH), jnp.float32),
  )(x, ws)


def _tc_block_update(agg, ws, base, n):
  nb = n // RB

  if base is None:
    def body(a0, a1, w_ref, o_ref):
      a = jnp.concatenate([a0[...], a1[...]], axis=1)
      o_ref[...] = jnp.maximum(lax.dot(a, w_ref[...]), 0.0)
    args = (agg, agg, ws)
    extra = []
  else:
    def body(a0, a1, w_ref, b_ref, o_ref):
      a = jnp.concatenate([a0[...], a1[...]], axis=1)
      o_ref[...] = jnp.maximum(lax.dot(a, w_ref[...]) + b_ref[...], 0.0)
    args = (agg, agg, ws, base)
    extra = [pl.BlockSpec((RB, CH), lambda c, b: (c * nb + b, 0))]

  return pl.pallas_call(
      body,
      grid=(NCORE, nb),
      in_specs=[
          pl.BlockSpec((RB, CH), lambda c, b: (b, 0)),
          pl.BlockSpec((RB, CH), lambda c, b: (nb + b, 0)),
          pl.BlockSpec((128, CH), lambda c, b: (c, 0)),
      ] + extra,
      out_specs=pl.BlockSpec((RB, CH), lambda c, b: (c * nb + b, 0)),
      out_shape=jax.ShapeDtypeStruct((NCORE * n, CH), jnp.float32),
  )(*args)


def _tc_out(hs, w_out):
  nb = NS // RB

  def body(h0, h1, w_ref, o_ref):
    a = jnp.concatenate([h0[...], h1[...]], axis=1)
    o_ref[...] = lax.dot(a, w_ref[...])

  return pl.pallas_call(
      body,
      grid=(nb,),
      in_specs=[
          pl.BlockSpec((RB, CH), lambda b: (b, 0)),
          pl.BlockSpec((RB, CH), lambda b: (nb + b, 0)),
          pl.BlockSpec((128, CH), lambda b: (0, 0)),
      ],
      out_specs=pl.BlockSpec((RB, CH), lambda b: (b, 0)),
      out_shape=jax.ShapeDtypeStruct((NS, CH), jnp.float32),
  )(hs, hs, w_out)


def kernel(surf_x, graph_x, bip_edge_weight, W_surf_in, W_graph_in,
           W_graph_blocks, W_graph2surf, W_surf2graph, W_out,
           graph_edge_index, bip_edge_index):
  sx = jnp.pad(surf_x, ((0, 0), (0, 3)))
  wsi = jnp.pad(W_surf_in, ((0, 3), (0, 0)))
  hs = _tc_embed(sx, _split_w(wsi), NS, 8)
  hg = _tc_embed(graph_x, _split_w(W_graph_in), NG, 128)

  ones_g = jnp.ones((EG,), jnp.float32)
  g_src, g_dst, g_w, epad_g = _pad_edges(
      graph_edge_index[0], graph_edge_index[1], ones_g, NG, NG)
  s_idx = bip_edge_index[0]
  g_idx = bip_edge_index[1]
  gs_src, gs_dst, gs_w, epad_b = _pad_edges(g_idx, s_idx, bip_edge_weight, NG, NS)
  sg_src, sg_dst, sg_w, _ = _pad_edges(s_idx, g_idx, bip_edge_weight, NS, NG)

  zk = jnp.zeros((K, CH), jnp.float32)
  z1 = jnp.zeros((K,), jnp.float32)

  agg_gg = _make_agg(NG, NG, epad_g, 1.0)
  agg_g2s = _make_agg(NG, NS, epad_b, 1e-6)
  agg_s2g = _make_agg(NS, NG, epad_b, 1e-6)


  for b in range(N_BLOCK):
    m = agg_gg(hg, g_src, g_dst, g_w, zk, z1)
    hg = _tc_block_update(m, _split_w(W_graph_blocks[b]), None, NG)
    m = agg_g2s(hg, gs_src, gs_dst, gs_w, zk, z1)
    hs = _tc_block_update(m, _split_w(W_graph2surf[b]), hs, NS)
    m = agg_s2g(hs, sg_src, sg_dst, sg_w, zk, z1)
    hg = _tc_block_update(m, _split_w(W_surf2graph[b]), hg, NG)

  return _tc_out(hs, W_out)


# speedup vs baseline: 7.8818x; 2.5217x over previous
import functools

import jax
import jax.numpy as jnp
from jax import lax
from jax.experimental import pallas as pl
from jax.experimental.pallas import tpu as pltpu
from jax.experimental.pallas import tpu_sc as plsc

NG = 10000
NS = 20000
EG = 320000
EB = 640000
N_BLOCK = 4
CH = 64
K = 128
NSUB = 16
NCORE = 2
RB = 1000


NCB = 8


def _pad_edges(src, dst, w, n_src, n_dst):
  e = src.shape[0]
  unit = NSUB * NCB * K * 2
  epad = ((e + unit - 1) // unit) * unit
  pad = epad - e
  ar = jnp.arange(pad, dtype=jnp.int32)
  src_p = jnp.concatenate([src.astype(jnp.int32), ar % n_src]).reshape(-1, NCB, K)
  dst_p = jnp.concatenate([dst.astype(jnp.int32), ar % n_dst]).reshape(-1, NCB, K)
  w_p = jnp.concatenate([w, jnp.zeros((pad,), jnp.float32)]).reshape(-1, NCB, K)
  return src_p, dst_p, w_p, epad


@functools.cache
def _make_agg(n_src, n_dst, epad, clip_lo):
  nblk = epad // (NSUB * NCB * K)
  assert nblk % 2 == 0
  per_sub = (n_dst // NSUB) & ~7
  rem = n_dst - NSUB * per_sub
  chunks = []
  off = 0
  while off < per_sub:
    chunks.append((off, min(K, per_sub - off)))
    off += min(K, per_sub - off)
  mesh = plsc.VectorSubcoreMesh(core_axis_name="c", subcore_axis_name="s")

  def body(x_hbm, src_hbm, dst_hbm, w_hbm, zk_hbm, z1_hbm, out_hbm,
           srcb0, srcb1, dstb0, dstb1, wb0, wb1,
           rows0, rows1, rows2, io_v, den_v, acc, acc1,
           sem_i0, sem_i1, sem_g0, sem_g1, sem_g2, sem_s0, sem_s1, sem_s2):
    cid = lax.axis_index("c")
    sid = lax.axis_index("s")
    row0 = pl.multiple_of(sid * per_sub, 8)
    srcb = (srcb0, srcb1)
    dstb = (dstb0, dstb1)
    wb = (wb0, wb1)
    rows = (rows0, rows1, rows2)
    sem_i = (sem_i0, sem_i1)
    sem_g = (sem_g0, sem_g1, sem_g2)
    sem_s = (sem_s0, sem_s1, sem_s2)

    pltpu.sync_copy(zk_hbm, io_v)
    pltpu.sync_copy(z1_hbm, den_v)
    for off, sz in chunks:
      pltpu.sync_copy(io_v.at[pl.ds(0, sz)], acc.at[pl.ds(row0 + off, sz)])
      pltpu.sync_copy(den_v.at[pl.ds(0, sz)], acc1.at[pl.ds(row0 + off, sz)])
    if rem:
      @pl.when(sid == 0)
      def _():
        base = NSUB * per_sub
        pltpu.sync_copy(io_v.at[pl.ds(0, rem)], acc.at[pl.ds(base, rem)])
        pltpu.sync_copy(den_v.at[pl.ds(0, rem)], acc1.at[pl.ds(base, rem)])
    plsc.subcore_barrier()

    coff = cid * n_src
    blk0 = sid * nblk

    def issue_idx(par, gblk):
      pltpu.async_copy(src_hbm.at[gblk], srcb[par], sem_i[par])
      pltpu.async_copy(dst_hbm.at[gblk], dstb[par], sem_i[par])
      pltpu.async_copy(w_hbm.at[gblk], wb[par], sem_i[par])

    def wait_idx(par):
      pltpu.make_async_copy(src_hbm.at[0], srcb[par], sem_i[par]).wait()
      pltpu.make_async_copy(dst_hbm.at[0], dstb[par], sem_i[par]).wait()
      pltpu.make_async_copy(w_hbm.at[0], wb[par], sem_i[par]).wait()

    def drain_scatters():
      for y in ((NCB - 3) % 3, (NCB - 2) % 3, (NCB - 1) % 3):
        pltpu.make_async_copy(zk_hbm, io_v, sem_s[y]).wait()

    def process_half(par):
      wait_idx(par)
      for j in range(NCB):
        for i in range(K // 16):
          sl = pl.ds(i * 16, 16)
          srcb[par][j, sl] = srcb[par][j, sl] + coff
      gath = {}
      for j in range(min(2, NCB)):
        gath[j] = pltpu.async_copy(
            x_hbm.at[srcb[par].at[j]], rows[j % 3], sem_g[j % 3])
      scat = {}
      for j in range(NCB):
        j3 = j % 3
        gath[j].wait()

        def scale_row(r, c2):
          wv = plsc.load_gather(
              wb[par], [jnp.full((16,), j, jnp.int32),
                        jnp.full((16,), r, jnp.int32)])
          for c in range(CH // 16):
            sl = pl.ds(c * 16, 16)
            rows[j3][r, sl] = rows[j3][r, sl] * wv
          return c2

        lax.fori_loop(0, K, scale_row, 0)
        if j + 2 < NCB:
          if j >= 1:
            scat[j - 1].wait()
          gath[j + 2] = pltpu.async_copy(
              x_hbm.at[srcb[par].at[j + 2]], rows[(j + 2) % 3],
              sem_g[(j + 2) % 3])
        pltpu.sync_copy(wb[par].at[j], acc1.at[dstb[par].at[j]], add=True)
        scat[j] = pltpu.async_copy(
            rows[j3], acc.at[dstb[par].at[j]], sem_s[j3], add=True)

    issue_idx(0, blk0)

    def blk_body(b2, carry):
      @pl.when(b2 > 0)
      def _():
        drain_scatters()
      issue_idx(1, blk0 + 2 * b2 + 1)
      process_half(0)
      drain_scatters()

      @pl.when(b2 + 1 < nblk // 2)
      def _():
        issue_idx(0, blk0 + 2 * b2 + 2)
      process_half(1)
      return carry

    lax.fori_loop(0, nblk // 2, blk_body, 0)
    drain_scatters()
    plsc.subcore_barrier()

    def emit_readout(r0, sz):
      pltpu.sync_copy(acc.at[pl.ds(r0, sz)], io_v.at[pl.ds(0, sz)])
      pltpu.sync_copy(acc1.at[pl.ds(r0, sz)], den_v.at[pl.ds(0, sz)])

      def div_row(r, c2):
        db = plsc.load_gather(den_v, [jnp.full((16,), r, jnp.int32)])
        inv = 1.0 / jnp.maximum(db, clip_lo)
        for c in range(CH // 16):
          sl = pl.ds(c * 16, 16)
          io_v[r, sl] = io_v[r, sl] * inv
        return c2

      lax.fori_loop(0, sz, div_row, 0)
      pltpu.sync_copy(io_v.at[pl.ds(0, sz)],
                      out_hbm.at[pl.ds(pl.multiple_of(cid * n_dst + r0, 8), sz)])

    for off, sz in chunks:
      emit_readout(row0 + off, sz)
    if rem:
      @pl.when(sid == 0)
      def _():
        emit_readout(NSUB * per_sub, rem)

  return pl.kernel(
      body,
      out_type=jax.ShapeDtypeStruct((NCORE * n_dst, CH), jnp.float32),
      mesh=mesh,
      compiler_params=pltpu.CompilerParams(
          needs_layout_passes=False, use_tc_tiling_on_sc=False),
      scratch_types=[
          pltpu.VMEM((NCB, K), jnp.int32),
          pltpu.VMEM((NCB, K), jnp.int32),
          pltpu.VMEM((NCB, K), jnp.int32),
          pltpu.VMEM((NCB, K), jnp.int32),
          pltpu.VMEM((NCB, K), jnp.float32),
          pltpu.VMEM((NCB, K), jnp.float32),
          pltpu.VMEM((K, CH), jnp.float32),
          pltpu.VMEM((K, CH), jnp.float32),
          pltpu.VMEM((K, CH), jnp.float32),
          pltpu.VMEM((K, CH), jnp.float32),
          pltpu.VMEM((K,), jnp.float32),
          pltpu.VMEM_SHARED((n_dst, CH), jnp.float32),
          pltpu.VMEM_SHARED((n_dst,), jnp.float32),
      ] + [pltpu.SemaphoreType.DMA] * 8)


def _split_w(w):
  kdim = w.shape[0]
  return jnp.swapaxes(w.reshape(kdim, NCORE, CH), 0, 1).reshape(NCORE * kdim, CH)


def _tc_embed(x, ws, n, kdim):
  nb = n // RB

  def body(x_ref, w_ref, o_ref):
    o_ref[...] = lax.dot(x_ref[...], w_ref[...])

  return pl.pallas_call(
      body,
      grid=(NCORE, nb),
      in_specs=[
          pl.BlockSpec((RB, kdim), lambda c, b: (b, 0)),
          pl.BlockSpec((kdim, CH), lambda c, b: (c, 0)),
      ],
      out_specs=pl.BlockSpec((RB, CH), lambda c, b: (c * nb + b, 0)),
      out_shape=jax.ShapeDtypeStruct((NCORE * n, CH), jnp.float32),
  )(x, ws)


def _tc_block_update(agg, ws, base, n):
  nb = n // RB

  if base is None:
    def body(a0, a1, w_ref, o_ref):
      a = jnp.concatenate([a0[...], a1[...]], axis=1)
      o_ref[...] = jnp.maximum(lax.dot(a, w_ref[...]), 0.0)
    args = (agg, agg, ws)
    extra = []
  else:
    def body(a0, a1, w_ref, b_ref, o_ref):
      a = jnp.concatenate([a0[...], a1[...]], axis=1)
      o_ref[...] = jnp.maximum(lax.dot(a, w_ref[...]) + b_ref[...], 0.0)
    args = (agg, agg, ws, base)
    extra = [pl.BlockSpec((RB, CH), lambda c, b: (c * nb + b, 0))]

  return pl.pallas_call(
      body,
      grid=(NCORE, nb),
      in_specs=[
          pl.BlockSpec((RB, CH), lambda c, b: (b, 0)),
          pl.BlockSpec((RB, CH), lambda c, b: (nb + b, 0)),
          pl.BlockSpec((128, CH), lambda c, b: (c, 0)),
      ] + extra,
      out_specs=pl.BlockSpec((RB, CH), lambda c, b: (c * nb + b, 0)),
      out_shape=jax.ShapeDtypeStruct((NCORE * n, CH), jnp.float32),
  )(*args)


def _tc_out(hs, w_out):
  nb = NS // RB

  def body(h0, h1, w_ref, o_ref):
    a = jnp.concatenate([h0[...], h1[...]], axis=1)
    o_ref[...] = lax.dot(a, w_ref[...])

  return pl.pallas_call(
      body,
      grid=(nb,),
      in_specs=[
          pl.BlockSpec((RB, CH), lambda b: (b, 0)),
          pl.BlockSpec((RB, CH), lambda b: (nb + b, 0)),
          pl.BlockSpec((128, CH), lambda b: (0, 0)),
      ],
      out_specs=pl.BlockSpec((RB, CH), lambda b: (b, 0)),
      out_shape=jax.ShapeDtypeStruct((NS, CH), jnp.float32),
  )(hs, hs, w_out)


def kernel(surf_x, graph_x, bip_edge_weight, W_surf_in, W_graph_in,
           W_graph_blocks, W_graph2surf, W_surf2graph, W_out,
           graph_edge_index, bip_edge_index):
  sx = jnp.pad(surf_x, ((0, 0), (0, 3)))
  wsi = jnp.pad(W_surf_in, ((0, 3), (0, 0)))
  hs = _tc_embed(sx, _split_w(wsi), NS, 8)
  hg = _tc_embed(graph_x, _split_w(W_graph_in), NG, 128)

  ones_g = jnp.ones((EG,), jnp.float32)
  g_src, g_dst, g_w, epad_g = _pad_edges(
      graph_edge_index[0], graph_edge_index[1], ones_g, NG, NG)
  s_idx = bip_edge_index[0]
  g_idx = bip_edge_index[1]
  gs_src, gs_dst, gs_w, epad_b = _pad_edges(g_idx, s_idx, bip_edge_weight, NG, NS)
  sg_src, sg_dst, sg_w, _ = _pad_edges(s_idx, g_idx, bip_edge_weight, NS, NG)

  zk = jnp.zeros((K, CH), jnp.float32)
  z1 = jnp.zeros((K,), jnp.float32)

  agg_gg = _make_agg(NG, NG, epad_g, 1.0)
  agg_g2s = _make_agg(NG, NS, epad_b, 1e-6)
  agg_s2g = _make_agg(NS, NG, epad_b, 1e-6)


  for b in range(N_BLOCK):
    m = agg_gg(hg, g_src, g_dst, g_w, zk, z1)
    hg = _tc_block_update(m, _split_w(W_graph_blocks[b]), None, NG)
    m = agg_g2s(hg, gs_src, gs_dst, gs_w, zk, z1)
    hs = _tc_block_update(m, _split_w(W_graph2surf[b]), hs, NS)
    m = agg_s2g(hs, sg_src, sg_dst, sg_w, zk, z1)
    hg = _tc_block_update(m, _split_w(W_surf2graph[b]), hg, NG)

  return _tc_out(hs, W_out)


# speedup vs baseline: 9.0927x; 1.1536x over previous
import functools

import jax
import jax.numpy as jnp
from jax import lax
from jax.experimental import pallas as pl
from jax.experimental.pallas import tpu as pltpu
from jax.experimental.pallas import tpu_sc as plsc

NG = 10000
NS = 20000
EG = 320000
EB = 640000
N_BLOCK = 4
CH = 64
K = 128
NSUB = 16
NCORE = 2
RB = 1000


NCB = 8


def _pad_edges(src, dst, w, n_src, n_dst, garbage_dst=False):
  e = src.shape[0]
  unit = NSUB * NCB * K * 2
  epad = ((e + unit - 1) // unit) * unit
  pad = epad - e
  ar = jnp.arange(pad, dtype=jnp.int32)
  dpad = n_dst + (ar % 8) if garbage_dst else ar % n_dst
  src_p = jnp.concatenate([src.astype(jnp.int32), ar % n_src]).reshape(-1, NCB, K)
  dst_p = jnp.concatenate([dst.astype(jnp.int32), dpad]).reshape(-1, NCB, K)
  w_p = jnp.concatenate([w, jnp.zeros((pad,), jnp.float32)]).reshape(-1, NCB, K)
  return src_p, dst_p, w_p, epad


@functools.cache
def _make_agg(n_src, n_dst, epad, clip_lo, weighted=True):
  nblk = epad // (NSUB * NCB * K)
  assert nblk % 2 == 0
  n_acc = n_dst if weighted else n_dst + 8
  per_sub = (n_dst // NSUB) & ~7
  rem = n_dst - NSUB * per_sub
  chunks = []
  off = 0
  while off < per_sub:
    chunks.append((off, min(K, per_sub - off)))
    off += min(K, per_sub - off)
  mesh = plsc.VectorSubcoreMesh(core_axis_name="c", subcore_axis_name="s")

  def body(x_hbm, src_hbm, dst_hbm, w_hbm, zk_hbm, z1_hbm, out_hbm,
           srcb0, srcb1, dstb0, dstb1, wb0, wb1,
           rows0, rows1, rows2, io_v, den_v, acc, acc1,
           sem_i0, sem_i1, sem_g0, sem_g1, sem_g2, sem_s0, sem_s1, sem_s2,
           sem_w0, sem_w1, sem_w2):
    cid = lax.axis_index("c")
    sid = lax.axis_index("s")
    row0 = pl.multiple_of(sid * per_sub, 8)
    srcb = (srcb0, srcb1)
    dstb = (dstb0, dstb1)
    wb = (wb0, wb1)
    rows = (rows0, rows1, rows2)
    sem_i = (sem_i0, sem_i1)
    sem_g = (sem_g0, sem_g1, sem_g2)
    sem_s = (sem_s0, sem_s1, sem_s2)
    sem_w = (sem_w0, sem_w1, sem_w2)

    pltpu.sync_copy(zk_hbm, io_v)
    pltpu.sync_copy(z1_hbm, den_v)
    for off, sz in chunks:
      pltpu.sync_copy(io_v.at[pl.ds(0, sz)], acc.at[pl.ds(row0 + off, sz)])
      pltpu.sync_copy(den_v.at[pl.ds(0, sz)], acc1.at[pl.ds(row0 + off, sz)])
    rem_z = rem + (n_acc - n_dst)
    if rem_z:
      @pl.when(sid == 0)
      def _():
        base = NSUB * per_sub
        pltpu.sync_copy(io_v.at[pl.ds(0, rem_z)], acc.at[pl.ds(base, rem_z)])
        pltpu.sync_copy(den_v.at[pl.ds(0, rem_z)], acc1.at[pl.ds(base, rem_z)])
    plsc.subcore_barrier()

    coff = cid * n_src
    blk0 = sid * nblk

    def issue_idx(par, gblk):
      pltpu.async_copy(src_hbm.at[gblk], srcb[par], sem_i[par])
      pltpu.async_copy(dst_hbm.at[gblk], dstb[par], sem_i[par])
      pltpu.async_copy(w_hbm.at[gblk], wb[par], sem_i[par])

    def wait_idx(par):
      pltpu.make_async_copy(src_hbm.at[0], srcb[par], sem_i[par]).wait()
      pltpu.make_async_copy(dst_hbm.at[0], dstb[par], sem_i[par]).wait()
      pltpu.make_async_copy(w_hbm.at[0], wb[par], sem_i[par]).wait()

    def drain_scatters():
      for y in ((NCB - 3) % 3, (NCB - 2) % 3, (NCB - 1) % 3):
        pltpu.make_async_copy(zk_hbm, io_v, sem_s[y]).wait()
        pltpu.make_async_copy(w_hbm.at[0, 0], den_v, sem_w[y]).wait()

    def process_half(par):
      wait_idx(par)
      for j in range(NCB):
        for i in range(K // 16):
          sl = pl.ds(i * 16, 16)
          srcb[par][j, sl] = srcb[par][j, sl] + coff
      gath = {}
      for j in range(min(2, NCB)):
        gath[j] = pltpu.async_copy(
            x_hbm.at[srcb[par].at[j]], rows[j % 3], sem_g[j % 3])
      scat = {}
      wscat = {}
      for j in range(NCB):
        j3 = j % 3
        gath[j].wait()

        if weighted:
          def scale_rows(r, c2):
            for u in range(2):
              wv = plsc.load_gather(
                  wb[par], [jnp.full((16,), j, jnp.int32),
                            jnp.full((16,), 2 * r + u, jnp.int32)])
              for c in range(CH // 16):
                sl = pl.ds(c * 16, 16)
                rows[j3][2 * r + u, sl] = rows[j3][2 * r + u, sl] * wv
            return c2

          lax.fori_loop(0, K // 2, scale_rows, 0)
        if j + 2 < NCB:
          if j >= 1:
            scat[j - 1].wait()
            wscat[j - 1].wait()
          gath[j + 2] = pltpu.async_copy(
              x_hbm.at[srcb[par].at[j + 2]], rows[(j + 2) % 3],
              sem_g[(j + 2) % 3])
        wscat[j] = pltpu.async_copy(
            wb[par].at[j], acc1.at[dstb[par].at[j]], sem_w[j3], add=True)
        scat[j] = pltpu.async_copy(
            rows[j3], acc.at[dstb[par].at[j]], sem_s[j3], add=True)

    issue_idx(0, blk0)

    def blk_body(b2, carry):
      @pl.when(b2 > 0)
      def _():
        drain_scatters()
      issue_idx(1, blk0 + 2 * b2 + 1)
      process_half(0)
      drain_scatters()

      @pl.when(b2 + 1 < nblk // 2)
      def _():
        issue_idx(0, blk0 + 2 * b2 + 2)
      process_half(1)
      return carry

    lax.fori_loop(0, nblk // 2, blk_body, 0)
    drain_scatters()
    plsc.subcore_barrier()

    def emit_readout(r0, sz):
      pltpu.sync_copy(acc.at[pl.ds(r0, sz)], io_v.at[pl.ds(0, sz)])
      pltpu.sync_copy(acc1.at[pl.ds(r0, sz)], den_v.at[pl.ds(0, sz)])

      def div_row(r, c2):
        db = plsc.load_gather(den_v, [jnp.full((16,), r, jnp.int32)])
        inv = 1.0 / jnp.maximum(db, clip_lo)
        for c in range(CH // 16):
          sl = pl.ds(c * 16, 16)
          io_v[r, sl] = io_v[r, sl] * inv
        return c2

      lax.fori_loop(0, sz, div_row, 0)
      pltpu.sync_copy(io_v.at[pl.ds(0, sz)],
                      out_hbm.at[pl.ds(pl.multiple_of(cid * n_dst + r0, 8), sz)])

    for off, sz in chunks:
      emit_readout(row0 + off, sz)
    if rem:
      @pl.when(sid == 0)
      def _():
        emit_readout(NSUB * per_sub, rem)

  return pl.kernel(
      body,
      out_type=jax.ShapeDtypeStruct((NCORE * n_dst, CH), jnp.float32),
      mesh=mesh,
      compiler_params=pltpu.CompilerParams(
          needs_layout_passes=False, use_tc_tiling_on_sc=False),
      scratch_types=[
          pltpu.VMEM((NCB, K), jnp.int32),
          pltpu.VMEM((NCB, K), jnp.int32),
          pltpu.VMEM((NCB, K), jnp.int32),
          pltpu.VMEM((NCB, K), jnp.int32),
          pltpu.VMEM((NCB, K), jnp.float32),
          pltpu.VMEM((NCB, K), jnp.float32),
          pltpu.VMEM((K, CH), jnp.float32),
          pltpu.VMEM((K, CH), jnp.float32),
          pltpu.VMEM((K, CH), jnp.float32),
          pltpu.VMEM((K, CH), jnp.float32),
          pltpu.VMEM((K,), jnp.float32),
          pltpu.VMEM_SHARED((n_acc, CH), jnp.float32),
          pltpu.VMEM_SHARED((n_acc,), jnp.float32),
      ] + [pltpu.SemaphoreType.DMA] * 11)


def _split_w(w):
  kdim = w.shape[0]
  return jnp.swapaxes(w.reshape(kdim, NCORE, CH), 0, 1).reshape(NCORE * kdim, CH)


def _tc_embed(x, ws, n, kdim):
  nb = n // RB

  def body(x_ref, w_ref, o_ref):
    o_ref[...] = lax.dot(x_ref[...], w_ref[...])

  return pl.pallas_call(
      body,
      grid=(NCORE, nb),
      in_specs=[
          pl.BlockSpec((RB, kdim), lambda c, b: (b, 0)),
          pl.BlockSpec((kdim, CH), lambda c, b: (c, 0)),
      ],
      out_specs=pl.BlockSpec((RB, CH), lambda c, b: (c * nb + b, 0)),
      out_shape=jax.ShapeDtypeStruct((NCORE * n, CH), jnp.float32),
  )(x, ws)


def _tc_block_update(agg, ws, base, n):
  nb = n // RB

  if base is None:
    def body(a0, a1, w_ref, o_ref):
      a = jnp.concatenate([a0[...], a1[...]], axis=1)
      o_ref[...] = jnp.maximum(lax.dot(a, w_ref[...]), 0.0)
    args = (agg, agg, ws)
    extra = []
  else:
    def body(a0, a1, w_ref, b_ref, o_ref):
      a = jnp.concatenate([a0[...], a1[...]], axis=1)
      o_ref[...] = jnp.maximum(lax.dot(a, w_ref[...]) + b_ref[...], 0.0)
    args = (agg, agg, ws, base)
    extra = [pl.BlockSpec((RB, CH), lambda c, b: (c * nb + b, 0))]

  return pl.pallas_call(
      body,
      grid=(NCORE, nb),
      in_specs=[
          pl.BlockSpec((RB, CH), lambda c, b: (b, 0)),
          pl.BlockSpec((RB, CH), lambda c, b: (nb + b, 0)),
          pl.BlockSpec((128, CH), lambda c, b: (c, 0)),
      ] + extra,
      out_specs=pl.BlockSpec((RB, CH), lambda c, b: (c * nb + b, 0)),
      out_shape=jax.ShapeDtypeStruct((NCORE * n, CH), jnp.float32),
  )(*args)


def _tc_out(hs, w_out):
  nb = NS // RB

  def body(h0, h1, w_ref, o_ref):
    a = jnp.concatenate([h0[...], h1[...]], axis=1)
    o_ref[...] = lax.dot(a, w_ref[...])

  return pl.pallas_call(
      body,
      grid=(nb,),
      in_specs=[
          pl.BlockSpec((RB, CH), lambda b: (b, 0)),
          pl.BlockSpec((RB, CH), lambda b: (nb + b, 0)),
          pl.BlockSpec((128, CH), lambda b: (0, 0)),
      ],
      out_specs=pl.BlockSpec((RB, CH), lambda b: (b, 0)),
      out_shape=jax.ShapeDtypeStruct((NS, CH), jnp.float32),
  )(hs, hs, w_out)


def kernel(surf_x, graph_x, bip_edge_weight, W_surf_in, W_graph_in,
           W_graph_blocks, W_graph2surf, W_surf2graph, W_out,
           graph_edge_index, bip_edge_index):
  sx = jnp.pad(surf_x, ((0, 0), (0, 3)))
  wsi = jnp.pad(W_surf_in, ((0, 3), (0, 0)))
  hs = _tc_embed(sx, _split_w(wsi), NS, 8)
  hg = _tc_embed(graph_x, _split_w(W_graph_in), NG, 128)

  ones_g = jnp.ones((EG,), jnp.float32)
  g_src, g_dst, g_w, epad_g = _pad_edges(
      graph_edge_index[0], graph_edge_index[1], ones_g, NG, NG,
      garbage_dst=True)
  s_idx = bip_edge_index[0]
  g_idx = bip_edge_index[1]
  gs_src, gs_dst, gs_w, epad_b = _pad_edges(g_idx, s_idx, bip_edge_weight, NG, NS)
  sg_src, sg_dst, sg_w, _ = _pad_edges(s_idx, g_idx, bip_edge_weight, NS, NG)

  zk = jnp.zeros((K, CH), jnp.float32)
  z1 = jnp.zeros((K,), jnp.float32)

  agg_gg = _make_agg(NG, NG, epad_g, 1.0, weighted=False)
  agg_g2s = _make_agg(NG, NS, epad_b, 1e-6)
  agg_s2g = _make_agg(NS, NG, epad_b, 1e-6)


  for b in range(N_BLOCK):
    m = agg_gg(hg, g_src, g_dst, g_w, zk, z1)
    hg = _tc_block_update(m, _split_w(W_graph_blocks[b]), None, NG)
    m = agg_g2s(hg, gs_src, gs_dst, gs_w, zk, z1)
    hs = _tc_block_update(m, _split_w(W_graph2surf[b]), hs, NS)
    m = agg_s2g(hs, sg_src, sg_dst, sg_w, zk, z1)
    hg = _tc_block_update(m, _split_w(W_surf2graph[b]), hg, NG)

  return _tc_out(hs, W_out)


# speedup vs baseline: 9.1405x; 1.0052x over previous
import functools

import jax
import jax.numpy as jnp
from jax import lax
from jax.experimental import pallas as pl
from jax.experimental.pallas import tpu as pltpu
from jax.experimental.pallas import tpu_sc as plsc

NG = 10000
NS = 20000
EG = 320000
EB = 640000
N_BLOCK = 4
CH = 64
K = 128
NSUB = 16
NCORE = 2
RB = 1000


NCB = 8


def _pad_edges(src, dst, w, n_src, n_dst, garbage_dst=False):
  e = src.shape[0]
  unit = NSUB * NCB * K * 2
  epad = ((e + unit - 1) // unit) * unit
  pad = epad - e
  ar = jnp.arange(pad, dtype=jnp.int32)
  dpad = n_dst + (ar % 8) if garbage_dst else ar % n_dst
  src_p = jnp.concatenate([src.astype(jnp.int32), ar % n_src]).reshape(-1, NCB, K)
  dst_p = jnp.concatenate([dst.astype(jnp.int32), dpad]).reshape(-1, NCB, K)
  w_p = jnp.concatenate([w, jnp.zeros((pad,), jnp.float32)]).reshape(-1, NCB, K)
  return src_p, dst_p, w_p, epad


@functools.cache
def _make_agg(n_src, n_dst, epad, clip_lo, weighted=True):
  nblk = epad // (NSUB * NCB * K)
  assert nblk % 2 == 0
  n_acc = n_dst if weighted else n_dst + 8
  per_sub = (n_dst // NSUB) & ~7
  rem = n_dst - NSUB * per_sub
  chunks = []
  off = 0
  while off < per_sub:
    chunks.append((off, min(K, per_sub - off)))
    off += min(K, per_sub - off)
  mesh = plsc.VectorSubcoreMesh(core_axis_name="c", subcore_axis_name="s")

  def body(x_hbm, src_hbm, dst_hbm, w_hbm, zk_hbm, z1_hbm, out_hbm,
           srcb0, srcb1, dstb0, dstb1, wb0, wb1,
           rows0, rows1, rows2, rows3, io_v, den_v, den2_v, acc, acc1,
           sem_i0, sem_i1, sem_g0, sem_g1, sem_g2, sem_g3,
           sem_s0, sem_s1, sem_s2, sem_s3, sem_w0, sem_w1, sem_w2, sem_w3):
    cid = lax.axis_index("c")
    sid = lax.axis_index("s")
    row0 = pl.multiple_of(sid * per_sub, 8)
    srcb = (srcb0, srcb1)
    dstb = (dstb0, dstb1)
    wb = (wb0, wb1)
    rows = (rows0, rows1, rows2, rows3)
    sem_i = (sem_i0, sem_i1)
    sem_g = (sem_g0, sem_g1, sem_g2, sem_g3)
    sem_s = (sem_s0, sem_s1, sem_s2, sem_s3)
    sem_w = (sem_w0, sem_w1, sem_w2, sem_w3)

    pltpu.sync_copy(zk_hbm, io_v)
    pltpu.sync_copy(z1_hbm, den_v)
    zh = []
    for off, sz in chunks:
      zh.append(pltpu.async_copy(
          io_v.at[pl.ds(0, sz)], acc.at[pl.ds(row0 + off, sz)], sem_g0))
      zh.append(pltpu.async_copy(
          den_v.at[pl.ds(0, sz)], acc1.at[pl.ds(row0 + off, sz)], sem_g1))
    rem_z = rem + (n_acc - n_dst)
    if rem_z:
      @pl.when(sid == 0)
      def _():
        base = NSUB * per_sub
        pltpu.sync_copy(io_v.at[pl.ds(0, rem_z)], acc.at[pl.ds(base, rem_z)])
        pltpu.sync_copy(den_v.at[pl.ds(0, rem_z)], acc1.at[pl.ds(base, rem_z)])
    for h in zh:
      h.wait()
    plsc.subcore_barrier()

    coff = cid * n_src
    blk0 = sid * nblk

    def issue_idx(par, gblk):
      pltpu.async_copy(src_hbm.at[gblk], srcb[par], sem_i[par])
      pltpu.async_copy(dst_hbm.at[gblk], dstb[par], sem_i[par])
      pltpu.async_copy(w_hbm.at[gblk], wb[par], sem_i[par])

    def wait_idx(par):
      pltpu.make_async_copy(src_hbm.at[0], srcb[par], sem_i[par]).wait()
      pltpu.make_async_copy(dst_hbm.at[0], dstb[par], sem_i[par]).wait()
      pltpu.make_async_copy(w_hbm.at[0], wb[par], sem_i[par]).wait()

    def drain_scatters():
      for d in range(NCB - 4, NCB):
        y = d % 4
        pltpu.make_async_copy(zk_hbm, io_v, sem_s[y]).wait()
        pltpu.make_async_copy(w_hbm.at[0, 0], den_v, sem_w[y]).wait()

    def process_half(par):
      wait_idx(par)
      for j in range(NCB):
        for i in range(K // 16):
          sl = pl.ds(i * 16, 16)
          srcb[par][j, sl] = srcb[par][j, sl] + coff
      gath = {}
      for j in range(min(3, NCB)):
        gath[j] = pltpu.async_copy(
            x_hbm.at[srcb[par].at[j]], rows[j % 4], sem_g[j % 4])
      scat = {}
      wscat = {}
      for j in range(NCB):
        j3 = j % 4
        gath[j].wait()

        if weighted:
          def scale_rows(r, c2):
            for u in range(2):
              wv = plsc.load_gather(
                  wb[par], [jnp.full((16,), j, jnp.int32),
                            jnp.full((16,), 2 * r + u, jnp.int32)])
              for c in range(CH // 16):
                sl = pl.ds(c * 16, 16)
                rows[j3][2 * r + u, sl] = rows[j3][2 * r + u, sl] * wv
            return c2

          lax.fori_loop(0, K // 2, scale_rows, 0)
        if j + 3 < NCB:
          if j >= 1:
            scat[j - 1].wait()
            wscat[j - 1].wait()
          gath[j + 3] = pltpu.async_copy(
              x_hbm.at[srcb[par].at[j + 3]], rows[(j + 3) % 4],
              sem_g[(j + 3) % 4])
        wscat[j] = pltpu.async_copy(
            wb[par].at[j], acc1.at[dstb[par].at[j]], sem_w[j3], add=True)
        scat[j] = pltpu.async_copy(
            rows[j3], acc.at[dstb[par].at[j]], sem_s[j3], add=True)

    issue_idx(0, blk0)

    def blk_body(b2, carry):
      @pl.when(b2 > 0)
      def _():
        drain_scatters()
      issue_idx(1, blk0 + 2 * b2 + 1)
      process_half(0)
      drain_scatters()

      @pl.when(b2 + 1 < nblk // 2)
      def _():
        issue_idx(0, blk0 + 2 * b2 + 2)
      process_half(1)
      return carry

    lax.fori_loop(0, nblk // 2, blk_body, 0)
    drain_scatters()
    plsc.subcore_barrier()

    rbufs = ((io_v, den_v), (rows0, den2_v))

    def do_div(p, sz):
      iob, denb = rbufs[p]

      def div_row(r, c2):
        db = plsc.load_gather(denb, [jnp.full((16,), r, jnp.int32)])
        inv = 1.0 / jnp.maximum(db, clip_lo)
        for c in range(CH // 16):
          sl = pl.ds(c * 16, 16)
          iob[r, sl] = iob[r, sl] * inv
        return c2

      lax.fori_loop(0, sz, div_row, 0)

    def issue_load(i):
      off, sz = chunks[i]
      p = i % 2
      r0 = row0 + off
      return (pltpu.async_copy(acc.at[pl.ds(r0, sz)],
                               rbufs[p][0].at[pl.ds(0, sz)], sem_g[p]),
              pltpu.async_copy(acc1.at[pl.ds(r0, sz)],
                               rbufs[p][1].at[pl.ds(0, sz)], sem_g[2 + p]))

    loads = {0: issue_load(0)}
    stores = {}
    nch = len(chunks)
    for i, (off, sz) in enumerate(chunks):
      p = i % 2
      for h in loads[i]:
        h.wait()
      if i + 1 < nch:
        if i >= 1:
          stores[i - 1].wait()
        loads[i + 1] = issue_load(i + 1)
      do_div(p, sz)
      stores[i] = pltpu.async_copy(
          rbufs[p][0].at[pl.ds(0, sz)],
          out_hbm.at[pl.ds(pl.multiple_of(cid * n_dst + row0 + off, 8), sz)],
          sem_s[p])
    for i in (nch - 2, nch - 1):
      if i >= 0:
        stores[i].wait()

    def emit_readout(r0, sz):
      pltpu.sync_copy(acc.at[pl.ds(r0, sz)], io_v.at[pl.ds(0, sz)])
      pltpu.sync_copy(acc1.at[pl.ds(r0, sz)], den_v.at[pl.ds(0, sz)])
      do_div(0, sz)
      pltpu.sync_copy(io_v.at[pl.ds(0, sz)],
                      out_hbm.at[pl.ds(pl.multiple_of(cid * n_dst + r0, 8), sz)])
    if rem:
      @pl.when(sid == 0)
      def _():
        emit_readout(NSUB * per_sub, rem)

  return pl.kernel(
      body,
      out_type=jax.ShapeDtypeStruct((NCORE * n_dst, CH), jnp.float32),
      mesh=mesh,
      compiler_params=pltpu.CompilerParams(
          needs_layout_passes=False, use_tc_tiling_on_sc=False),
      scratch_types=[
          pltpu.VMEM((NCB, K), jnp.int32),
          pltpu.VMEM((NCB, K), jnp.int32),
          pltpu.VMEM((NCB, K), jnp.int32),
          pltpu.VMEM((NCB, K), jnp.int32),
          pltpu.VMEM((NCB, K), jnp.float32),
          pltpu.VMEM((NCB, K), jnp.float32),
          pltpu.VMEM((K, CH), jnp.float32),
          pltpu.VMEM((K, CH), jnp.float32),
          pltpu.VMEM((K, CH), jnp.float32),
          pltpu.VMEM((K, CH), jnp.float32),
          pltpu.VMEM((K, CH), jnp.float32),
          pltpu.VMEM((K,), jnp.float32),
          pltpu.VMEM((K,), jnp.float32),
          pltpu.VMEM_SHARED((n_acc, CH), jnp.float32),
          pltpu.VMEM_SHARED((n_acc,), jnp.float32),
      ] + [pltpu.SemaphoreType.DMA] * 14)


def _split_w(w):
  kdim = w.shape[0]
  return jnp.swapaxes(w.reshape(kdim, NCORE, CH), 0, 1).reshape(NCORE * kdim, CH)


def _tc_embed(x, ws, n, kdim):
  nb = n // RB

  def body(x_ref, w_ref, o_ref):
    o_ref[...] = lax.dot(x_ref[...], w_ref[...])

  return pl.pallas_call(
      body,
      grid=(NCORE, nb),
      in_specs=[
          pl.BlockSpec((RB, kdim), lambda c, b: (b, 0)),
          pl.BlockSpec((kdim, CH), lambda c, b: (c, 0)),
      ],
      out_specs=pl.BlockSpec((RB, CH), lambda c, b: (c * nb + b, 0)),
      out_shape=jax.ShapeDtypeStruct((NCORE * n, CH), jnp.float32),
  )(x, ws)


def _tc_block_update(agg, ws, base, n):
  nb = n // RB

  if base is None:
    def body(a0, a1, w_ref, o_ref):
      a = jnp.concatenate([a0[...], a1[...]], axis=1)
      o_ref[...] = jnp.maximum(lax.dot(a, w_ref[...]), 0.0)
    args = (agg, agg, ws)
    extra = []
  else:
    def body(a0, a1, w_ref, b_ref, o_ref):
      a = jnp.concatenate([a0[...], a1[...]], axis=1)
      o_ref[...] = jnp.maximum(lax.dot(a, w_ref[...]) + b_ref[...], 0.0)
    args = (agg, agg, ws, base)
    extra = [pl.BlockSpec((RB, CH), lambda c, b: (c * nb + b, 0))]

  return pl.pallas_call(
      body,
      grid=(NCORE, nb),
      in_specs=[
          pl.BlockSpec((RB, CH), lambda c, b: (b, 0)),
          pl.BlockSpec((RB, CH), lambda c, b: (nb + b, 0)),
          pl.BlockSpec((128, CH), lambda c, b: (c, 0)),
      ] + extra,
      out_specs=pl.BlockSpec((RB, CH), lambda c, b: (c * nb + b, 0)),
      out_shape=jax.ShapeDtypeStruct((NCORE * n, CH), jnp.float32),
  )(*args)


def _tc_out(hs, w_out):
  nb = NS // RB

  def body(h0, h1, w_ref, o_ref):
    a = jnp.concatenate([h0[...], h1[...]], axis=1)
    o_ref[...] = lax.dot(a, w_ref[...])

  return pl.pallas_call(
      body,
      grid=(nb,),
      in_specs=[
          pl.BlockSpec((RB, CH), lambda b: (b, 0)),
          pl.BlockSpec((RB, CH), lambda b: (nb + b, 0)),
          pl.BlockSpec((128, CH), lambda b: (0, 0)),
      ],
      out_specs=pl.BlockSpec((RB, CH), lambda b: (b, 0)),
      out_shape=jax.ShapeDtypeStruct((NS, CH), jnp.float32),
  )(hs, hs, w_out)


def kernel(surf_x, graph_x, bip_edge_weight, W_surf_in, W_graph_in,
           W_graph_blocks, W_graph2surf, W_surf2graph, W_out,
           graph_edge_index, bip_edge_index):
  sx = jnp.pad(surf_x, ((0, 0), (0, 3)))
  wsi = jnp.pad(W_surf_in, ((0, 3), (0, 0)))
  hs = _tc_embed(sx, _split_w(wsi), NS, 8)
  hg = _tc_embed(graph_x, _split_w(W_graph_in), NG, 128)

  ones_g = jnp.ones((EG,), jnp.float32)
  g_src, g_dst, g_w, epad_g = _pad_edges(
      graph_edge_index[0], graph_edge_index[1], ones_g, NG, NG,
      garbage_dst=True)
  s_idx = bip_edge_index[0]
  g_idx = bip_edge_index[1]
  gs_src, gs_dst, gs_w, epad_b = _pad_edges(g_idx, s_idx, bip_edge_weight, NG, NS)
  sg_src, sg_dst, sg_w, _ = _pad_edges(s_idx, g_idx, bip_edge_weight, NS, NG)

  zk = jnp.zeros((K, CH), jnp.float32)
  z1 = jnp.zeros((K,), jnp.float32)

  agg_gg = _make_agg(NG, NG, epad_g, 1.0, weighted=False)
  agg_g2s = _make_agg(NG, NS, epad_b, 1e-6)
  agg_s2g = _make_agg(NS, NG, epad_b, 1e-6)


  for b in range(N_BLOCK):
    m = agg_gg(hg, g_src, g_dst, g_w, zk, z1)
    hg = _tc_block_update(m, _split_w(W_graph_blocks[b]), None, NG)
    m = agg_g2s(hg, gs_src, gs_dst, gs_w, zk, z1)
    hs = _tc_block_update(m, _split_w(W_graph2surf[b]), hs, NS)
    m = agg_s2g(hs, sg_src, sg_dst, sg_w, zk, z1)
    hg = _tc_block_update(m, _split_w(W_surf2graph[b]), hg, NG)

  return _tc_out(hs, W_out)


# speedup vs baseline: 11.3552x; 1.2423x over previous
import functools

import jax
import jax.numpy as jnp
from jax import lax
from jax.experimental import pallas as pl
from jax.experimental.pallas import tpu as pltpu
from jax.experimental.pallas import tpu_sc as plsc

NG = 10000
NS = 20000
EG = 320000
EB = 640000
N_BLOCK = 4
CH = 64
K = 128
NSUB = 16
NCORE = 2
RB = 1000


NCB = 8


def _pad_edges(src, dst, w, n_src, n_dst, garbage_dst=False):
  e = src.shape[0]
  unit = NSUB * NCB * K * 2
  epad = ((e + unit - 1) // unit) * unit
  pad = epad - e
  ar = jnp.arange(pad, dtype=jnp.int32)
  dpad = n_dst + (ar % 8) if garbage_dst else ar % n_dst
  src_p = jnp.concatenate([src.astype(jnp.int32), ar % n_src]).reshape(-1, NCB, K)
  dst_p = jnp.concatenate([dst.astype(jnp.int32), dpad]).reshape(-1, NCB, K)
  w_p = jnp.concatenate([w, jnp.zeros((pad,), jnp.float32)]).reshape(-1, NCB, K)
  return src_p, dst_p, w_p, epad


@functools.cache
def _make_agg(n_src, n_dst, epad, clip_lo, weighted=True):
  nblk = epad // (NSUB * NCB * K)
  assert nblk % 2 == 0
  n_acc = n_dst if weighted else n_dst + 8
  per_sub = (n_dst // NSUB) & ~7
  rem = n_dst - NSUB * per_sub
  chunks = []
  off = 0
  while off < per_sub:
    chunks.append((off, min(K, per_sub - off)))
    off += min(K, per_sub - off)
  mesh = plsc.VectorSubcoreMesh(core_axis_name="c", subcore_axis_name="s")

  def body(x_hbm, src_hbm, dst_hbm, w_hbm, zk_hbm, z1_hbm, out_hbm,
           srcb0, srcb1, dstb0, dstb1, wb0, wb1,
           rows0, rows1, rows2, rows3, io_v, den_v, den2_v, acc, acc1,
           sem_i0, sem_i1, sem_g0, sem_g1, sem_g2, sem_g3,
           sem_s0, sem_s1, sem_s2, sem_s3, sem_w0, sem_w1, sem_w2, sem_w3):
    cid = lax.axis_index("c")
    sid = lax.axis_index("s")
    row0 = pl.multiple_of(sid * per_sub, 8)
    srcb = (srcb0, srcb1)
    dstb = (dstb0, dstb1)
    wb = (wb0, wb1)
    rows = (rows0, rows1, rows2, rows3)
    sem_i = (sem_i0, sem_i1)
    sem_g = (sem_g0, sem_g1, sem_g2, sem_g3)
    sem_s = (sem_s0, sem_s1, sem_s2, sem_s3)
    sem_w = (sem_w0, sem_w1, sem_w2, sem_w3)

    pltpu.sync_copy(zk_hbm, io_v)
    pltpu.sync_copy(z1_hbm, den_v)
    zh = []
    for off, sz in chunks:
      zh.append(pltpu.async_copy(
          io_v.at[pl.ds(0, sz)], acc.at[pl.ds(row0 + off, sz)], sem_g0))
      zh.append(pltpu.async_copy(
          den_v.at[pl.ds(0, sz)], acc1.at[pl.ds(row0 + off, sz)], sem_g1))
    rem_z = rem + (n_acc - n_dst)
    if rem_z:
      @pl.when(sid == 0)
      def _():
        base = NSUB * per_sub
        pltpu.sync_copy(io_v.at[pl.ds(0, rem_z)], acc.at[pl.ds(base, rem_z)])
        pltpu.sync_copy(den_v.at[pl.ds(0, rem_z)], acc1.at[pl.ds(base, rem_z)])
    for h in zh:
      h.wait()
    plsc.subcore_barrier()

    coff = cid * n_src
    blk0 = sid * nblk

    def issue_idx(par, gblk):
      pltpu.async_copy(src_hbm.at[gblk], srcb[par], sem_i[par])
      pltpu.async_copy(dst_hbm.at[gblk], dstb[par], sem_i[par])
      pltpu.async_copy(w_hbm.at[gblk], wb[par], sem_i[par])

    def wait_idx(par):
      pltpu.make_async_copy(src_hbm.at[0], srcb[par], sem_i[par]).wait()
      pltpu.make_async_copy(dst_hbm.at[0], dstb[par], sem_i[par]).wait()
      pltpu.make_async_copy(w_hbm.at[0], wb[par], sem_i[par]).wait()

    def drain_scatters():
      for d in range(NCB - 4, NCB):
        y = d % 4
        pltpu.make_async_copy(zk_hbm, io_v, sem_s[y]).wait()
        pltpu.make_async_copy(w_hbm.at[0, 0], den_v, sem_w[y]).wait()

    def process_half(par):
      wait_idx(par)
      for j in range(NCB):
        for i in range(K // 16):
          sl = pl.ds(i * 16, 16)
          srcb[par][j, sl] = srcb[par][j, sl] + coff
      gath = {}
      for j in range(min(3, NCB)):
        gath[j] = pltpu.async_copy(
            x_hbm.at[srcb[par].at[j]], rows[j % 4], sem_g[j % 4])
      scat = {}
      wscat = {}
      for j in range(NCB):
        j3 = j % 4
        gath[j].wait()

        if weighted:
          jv = jnp.full((16,), j, jnp.int32)

          @plsc.parallel_loop(0, K // 4, unroll=2)
          def _(r4):
            for u in range(4):
              r = 4 * r4 + u
              wv = plsc.load_gather(
                  wb[par], [jv, jnp.full((16,), r, jnp.int32)])
              for c in range(CH // 16):
                sl = pl.ds(c * 16, 16)
                rows[j3][r, sl] = rows[j3][r, sl] * wv
        if j + 3 < NCB:
          if j >= 1:
            scat[j - 1].wait()
            wscat[j - 1].wait()
          gath[j + 3] = pltpu.async_copy(
              x_hbm.at[srcb[par].at[j + 3]], rows[(j + 3) % 4],
              sem_g[(j + 3) % 4])
        wscat[j] = pltpu.async_copy(
            wb[par].at[j], acc1.at[dstb[par].at[j]], sem_w[j3], add=True)
        scat[j] = pltpu.async_copy(
            rows[j3], acc.at[dstb[par].at[j]], sem_s[j3], add=True)

    issue_idx(0, blk0)

    def blk_body(b2, carry):
      @pl.when(b2 > 0)
      def _():
        drain_scatters()
      issue_idx(1, blk0 + 2 * b2 + 1)
      process_half(0)
      drain_scatters()

      @pl.when(b2 + 1 < nblk // 2)
      def _():
        issue_idx(0, blk0 + 2 * b2 + 2)
      process_half(1)
      return carry

    lax.fori_loop(0, nblk // 2, blk_body, 0)
    drain_scatters()
    plsc.subcore_barrier()

    rbufs = ((io_v, den_v), (rows0, den2_v))

    def do_div(p, sz):
      iob, denb = rbufs[p]

      def div_row(r, c2):
        db = plsc.load_gather(denb, [jnp.full((16,), r, jnp.int32)])
        inv = 1.0 / jnp.maximum(db, clip_lo)
        for c in range(CH // 16):
          sl = pl.ds(c * 16, 16)
          iob[r, sl] = iob[r, sl] * inv
        return c2

      lax.fori_loop(0, sz, div_row, 0)

    def issue_load(i):
      off, sz = chunks[i]
      p = i % 2
      r0 = row0 + off
      return (pltpu.async_copy(acc.at[pl.ds(r0, sz)],
                               rbufs[p][0].at[pl.ds(0, sz)], sem_g[p]),
              pltpu.async_copy(acc1.at[pl.ds(r0, sz)],
                               rbufs[p][1].at[pl.ds(0, sz)], sem_g[2 + p]))

    loads = {0: issue_load(0)}
    stores = {}
    nch = len(chunks)
    for i, (off, sz) in enumerate(chunks):
      p = i % 2
      for h in loads[i]:
        h.wait()
      if i + 1 < nch:
        if i >= 1:
          stores[i - 1].wait()
        loads[i + 1] = issue_load(i + 1)
      do_div(p, sz)
      stores[i] = pltpu.async_copy(
          rbufs[p][0].at[pl.ds(0, sz)],
          out_hbm.at[pl.ds(pl.multiple_of(cid * n_dst + row0 + off, 8), sz)],
          sem_s[p])
    for i in (nch - 2, nch - 1):
      if i >= 0:
        stores[i].wait()

    def emit_readout(r0, sz):
      pltpu.sync_copy(acc.at[pl.ds(r0, sz)], io_v.at[pl.ds(0, sz)])
      pltpu.sync_copy(acc1.at[pl.ds(r0, sz)], den_v.at[pl.ds(0, sz)])
      do_div(0, sz)
      pltpu.sync_copy(io_v.at[pl.ds(0, sz)],
                      out_hbm.at[pl.ds(pl.multiple_of(cid * n_dst + r0, 8), sz)])
    if rem:
      @pl.when(sid == 0)
      def _():
        emit_readout(NSUB * per_sub, rem)

  return pl.kernel(
      body,
      out_type=jax.ShapeDtypeStruct((NCORE * n_dst, CH), jnp.float32),
      mesh=mesh,
      compiler_params=pltpu.CompilerParams(
          needs_layout_passes=False, use_tc_tiling_on_sc=False),
      scratch_types=[
          pltpu.VMEM((NCB, K), jnp.int32),
          pltpu.VMEM((NCB, K), jnp.int32),
          pltpu.VMEM((NCB, K), jnp.int32),
          pltpu.VMEM((NCB, K), jnp.int32),
          pltpu.VMEM((NCB, K), jnp.float32),
          pltpu.VMEM((NCB, K), jnp.float32),
          pltpu.VMEM((K, CH), jnp.float32),
          pltpu.VMEM((K, CH), jnp.float32),
          pltpu.VMEM((K, CH), jnp.float32),
          pltpu.VMEM((K, CH), jnp.float32),
          pltpu.VMEM((K, CH), jnp.float32),
          pltpu.VMEM((K,), jnp.float32),
          pltpu.VMEM((K,), jnp.float32),
          pltpu.VMEM_SHARED((n_acc, CH), jnp.float32),
          pltpu.VMEM_SHARED((n_acc,), jnp.float32),
      ] + [pltpu.SemaphoreType.DMA] * 14)


def _split_w(w):
  kdim = w.shape[0]
  return jnp.swapaxes(w.reshape(kdim, NCORE, CH), 0, 1).reshape(NCORE * kdim, CH)


def _tc_embed(x, ws, n, kdim):
  nb = n // RB

  def body(x_ref, w_ref, o_ref):
    o_ref[...] = lax.dot(x_ref[...], w_ref[...])

  return pl.pallas_call(
      body,
      grid=(NCORE, nb),
      in_specs=[
          pl.BlockSpec((RB, kdim), lambda c, b: (b, 0)),
          pl.BlockSpec((kdim, CH), lambda c, b: (c, 0)),
      ],
      out_specs=pl.BlockSpec((RB, CH), lambda c, b: (c * nb + b, 0)),
      out_shape=jax.ShapeDtypeStruct((NCORE * n, CH), jnp.float32),
  )(x, ws)


def _tc_block_update(agg, ws, base, n):
  nb = n // RB

  if base is None:
    def body(a0, a1, w_ref, o_ref):
      a = jnp.concatenate([a0[...], a1[...]], axis=1)
      o_ref[...] = jnp.maximum(lax.dot(a, w_ref[...]), 0.0)
    args = (agg, agg, ws)
    extra = []
  else:
    def body(a0, a1, w_ref, b_ref, o_ref):
      a = jnp.concatenate([a0[...], a1[...]], axis=1)
      o_ref[...] = jnp.maximum(lax.dot(a, w_ref[...]) + b_ref[...], 0.0)
    args = (agg, agg, ws, base)
    extra = [pl.BlockSpec((RB, CH), lambda c, b: (c * nb + b, 0))]

  return pl.pallas_call(
      body,
      grid=(NCORE, nb),
      in_specs=[
          pl.BlockSpec((RB, CH), lambda c, b: (b, 0)),
          pl.BlockSpec((RB, CH), lambda c, b: (nb + b, 0)),
          pl.BlockSpec((128, CH), lambda c, b: (c, 0)),
      ] + extra,
      out_specs=pl.BlockSpec((RB, CH), lambda c, b: (c * nb + b, 0)),
      out_shape=jax.ShapeDtypeStruct((NCORE * n, CH), jnp.float32),
  )(*args)


def _tc_out(hs, w_out):
  nb = NS // RB

  def body(h0, h1, w_ref, o_ref):
    a = jnp.concatenate([h0[...], h1[...]], axis=1)
    o_ref[...] = lax.dot(a, w_ref[...])

  return pl.pallas_call(
      body,
      grid=(nb,),
      in_specs=[
          pl.BlockSpec((RB, CH), lambda b: (b, 0)),
          pl.BlockSpec((RB, CH), lambda b: (nb + b, 0)),
          pl.BlockSpec((128, CH), lambda b: (0, 0)),
      ],
      out_specs=pl.BlockSpec((RB, CH), lambda b: (b, 0)),
      out_shape=jax.ShapeDtypeStruct((NS, CH), jnp.float32),
  )(hs, hs, w_out)


def kernel(surf_x, graph_x, bip_edge_weight, W_surf_in, W_graph_in,
           W_graph_blocks, W_graph2surf, W_surf2graph, W_out,
           graph_edge_index, bip_edge_index):
  sx = jnp.pad(surf_x, ((0, 0), (0, 3)))
  wsi = jnp.pad(W_surf_in, ((0, 3), (0, 0)))
  hs = _tc_embed(sx, _split_w(wsi), NS, 8)
  hg = _tc_embed(graph_x, _split_w(W_graph_in), NG, 128)

  ones_g = jnp.ones((EG,), jnp.float32)
  g_src, g_dst, g_w, epad_g = _pad_edges(
      graph_edge_index[0], graph_edge_index[1], ones_g, NG, NG,
      garbage_dst=True)
  s_idx = bip_edge_index[0]
  g_idx = bip_edge_index[1]
  gs_src, gs_dst, gs_w, epad_b = _pad_edges(g_idx, s_idx, bip_edge_weight, NG, NS)
  sg_src, sg_dst, sg_w, _ = _pad_edges(s_idx, g_idx, bip_edge_weight, NS, NG)

  zk = jnp.zeros((K, CH), jnp.float32)
  z1 = jnp.zeros((K,), jnp.float32)

  agg_gg = _make_agg(NG, NG, epad_g, 1.0, weighted=False)
  agg_g2s = _make_agg(NG, NS, epad_b, 1e-6)
  agg_s2g = _make_agg(NS, NG, epad_b, 1e-6)


  for b in range(N_BLOCK):
    m = agg_gg(hg, g_src, g_dst, g_w, zk, z1)
    hg = _tc_block_update(m, _split_w(W_graph_blocks[b]), None, NG)
    m = agg_g2s(hg, gs_src, gs_dst, gs_w, zk, z1)
    hs = _tc_block_update(m, _split_w(W_graph2surf[b]), hs, NS)
    m = agg_s2g(hs, sg_src, sg_dst, sg_w, zk, z1)
    hg = _tc_block_update(m, _split_w(W_surf2graph[b]), hg, NG)

  return _tc_out(hs, W_out)
